# baseline (speedup 1.0000x reference)
import functools

import jax
import jax.numpy as jnp
from jax.experimental import pallas as pl
from jax.experimental.pallas import tpu as pltpu

_FANOUT = 4


def _gather_sum_tile(idx_ref, src, buf, base, m):
    for mi in range(m):
        o = base + _FANOUT * mi
        acc = None
        for k in range(_FANOUT):
            slab = src[pl.ds(idx_ref[o + k], 1), :]
            acc = slab if acc is None else acc + slab
        buf[pl.ds(mi, 1), :] = acc
    return buf[0:m, :]


def _l0_kernel(idx_ref, feat_hbm, w_ref, b_ref, o_ref, fbuf, buf, sem,
               *, m, nt):
    j = pl.program_id(1)

    @pl.when(j == 0)
    def _copy_src():
        cp = pltpu.make_async_copy(feat_hbm, fbuf, sem)
        cp.start()
        cp.wait()

    t = pl.program_id(0) * nt + j
    x = _gather_sum_tile(idx_ref, fbuf, buf, t * (m * _FANOUT), m)
    y = jnp.dot(x, w_ref[...], preferred_element_type=jnp.float32) + b_ref[...]
    o_ref[...] = jnp.maximum(y, 0.0)


def _l1_kernel(idx_ref, src_ref, w1_ref, wa_ref, wb_ref, b1_ref, o_ref, buf,
               *, m, nt):
    t = pl.program_id(0) * nt + pl.program_id(1)
    x = _gather_sum_tile(idx_ref, src_ref, buf, t * (m * _FANOUT), m)
    y = jnp.dot(x, w1_ref[...], preferred_element_type=jnp.float32) + b1_ref[...]
    yr = jnp.maximum(y, 0.0)
    o_ref[...] = (jnp.dot(y, wa_ref[...], preferred_element_type=jnp.float32)
                  + jnp.dot(yr, wb_ref[...], preferred_element_type=jnp.float32))


def _l2_kernel(idx_ref, src_ref, b2_ref, o_ref, *, m, nt):
    t = pl.program_id(0) * nt + pl.program_id(1)
    base = t * (m * _FANOUT)
    bias = b2_ref[...]
    for mi in range(m):
        o = base + _FANOUT * mi
        acc = (src_ref[pl.ds(idx_ref[o], 1), :]
               + src_ref[pl.ds(idx_ref[o + 1], 1), :]
               + src_ref[pl.ds(idx_ref[o + 2], 1), :]
               + src_ref[pl.ds(idx_ref[o + 3], 1), :])
        o_ref[pl.ds(mi, 1), :] = acc + bias


def _layer0(feat, idx, w, b, *, m):
    n_dst = idx.shape[0] // _FANOUT
    fin, fout = w.shape
    nt = n_dst // (2 * m)
    kern = functools.partial(_l0_kernel, m=m, nt=nt)
    return pl.pallas_call(
        kern,
        out_shape=jax.ShapeDtypeStruct((n_dst, fout), jnp.float32),
        grid_spec=pltpu.PrefetchScalarGridSpec(
            num_scalar_prefetch=1,
            grid=(2, nt),
            in_specs=[
                pl.BlockSpec(memory_space=pl.ANY),
                pl.BlockSpec((fin, fout), lambda i, j, idx: (0, 0)),
                pl.BlockSpec((1, fout), lambda i, j, idx: (0, 0)),
            ],
            out_specs=pl.BlockSpec((m, fout), lambda i, j, idx: (i * nt + j, 0)),
            scratch_shapes=[
                pltpu.VMEM(feat.shape, jnp.float32),
                pltpu.VMEM((m, fin), jnp.float32),
                pltpu.SemaphoreType.DMA,
            ],
        ),
        compiler_params=pltpu.CompilerParams(
            dimension_semantics=("parallel", "arbitrary"),
            vmem_limit_bytes=48 << 20,
        ),
    )(idx, feat, w, b)


def _layer1(src, idx, w1, wa, wb, b1, *, m):
    n_dst = idx.shape[0] // _FANOUT
    fin, fmid = w1.shape
    fout = wa.shape[1]
    nt = n_dst // (2 * m)
    kern = functools.partial(_l1_kernel, m=m, nt=nt)
    return pl.pallas_call(
        kern,
        out_shape=jax.ShapeDtypeStruct((n_dst, fout), jnp.float32),
        grid_spec=pltpu.PrefetchScalarGridSpec(
            num_scalar_prefetch=1,
            grid=(2, nt),
            in_specs=[
                pl.BlockSpec(src.shape, lambda i, j, idx: (0, 0)),
                pl.BlockSpec((fin, fmid), lambda i, j, idx: (0, 0)),
                pl.BlockSpec((fmid, fout), lambda i, j, idx: (0, 0)),
                pl.BlockSpec((fmid, fout), lambda i, j, idx: (0, 0)),
                pl.BlockSpec((1, fmid), lambda i, j, idx: (0, 0)),
            ],
            out_specs=pl.BlockSpec((m, fout), lambda i, j, idx: (i * nt + j, 0)),
            scratch_shapes=[
                pltpu.VMEM((m, fin), jnp.float32),
            ],
        ),
        compiler_params=pltpu.CompilerParams(
            dimension_semantics=("parallel", "arbitrary"),
            vmem_limit_bytes=40 << 20,
        ),
    )(idx, src, w1, wa, wb, b1)


def _layer2(src, idx, b2, *, m):
    n_dst = idx.shape[0] // _FANOUT
    fout = src.shape[-1]
    nt = n_dst // (2 * m)
    kern = functools.partial(_l2_kernel, m=m, nt=nt)
    return pl.pallas_call(
        kern,
        out_shape=jax.ShapeDtypeStruct((n_dst, fout), jnp.float32),
        grid_spec=pltpu.PrefetchScalarGridSpec(
            num_scalar_prefetch=1,
            grid=(2, nt),
            in_specs=[
                pl.BlockSpec(src.shape, lambda i, j, idx: (0, 0)),
                pl.BlockSpec((1, fout), lambda i, j, idx: (0, 0)),
            ],
            out_specs=pl.BlockSpec((m, fout), lambda i, j, idx: (i * nt + j, 0)),
            scratch_shapes=[],
        ),
        compiler_params=pltpu.CompilerParams(
            dimension_semantics=("parallel", "arbitrary"),
            vmem_limit_bytes=16 << 20,
        ),
    )(idx, src, b2)


def kernel(features, w0, b0, w1, b1, w2, b2, nbr0, nbr1, nbr2):
    f32 = jnp.float32
    fmid = w1.shape[0]

    idx0 = nbr0.astype(jnp.int32).reshape(-1)
    w0s = (w0.astype(f32) / _FANOUT)
    h1 = _layer0(features.astype(f32), idx0, w0s,
                 b0.astype(f32).reshape(1, -1), m=256)

    idx1 = nbr1.astype(jnp.int32).reshape(-1)
    w1s = (w1.astype(f32) / _FANOUT)
    wa = (w2[:fmid].astype(f32) / _FANOUT)
    wb = (w2[fmid:].astype(f32) / _FANOUT)
    z = _layer1(h1, idx1, w1s, wa, wb, b1.astype(f32).reshape(1, -1), m=256)

    idx2 = nbr2.astype(jnp.int32).reshape(-1)
    out = _layer2(z, idx2, b2.astype(f32).reshape(1, -1), m=256)
    return out.astype(f32)


# speedup vs baseline: 1.3821x; 1.3821x over previous
import functools

import jax
import jax.numpy as jnp
from jax import lax
from jax.experimental import pallas as pl
from jax.experimental.pallas import tpu as pltpu

_FANOUT = 4
_CHUNK = 2048


def _gather_sum_tile(idx_ref, src, buf, base, m, p):
    S = m + 1
    for mi in range(m):
        o = base + _FANOUT * mi
        acc = None
        for k in range(_FANOUT):
            ik = pl.multiple_of(idx_ref[o + k], p)
            slab = src[pl.ds(ik, p), :]
            acc = slab if acc is None else acc + slab
        buf[mi:mi + p * S:S, :] = acc
    return jnp.concatenate([buf[c * S:c * S + m, :] for c in range(p)],
                           axis=-1)


def _l0_kernel(idx_ref, feat_hbm, w_ref, b_ref, o_ref, fbuf, tmp0, tmp1, buf,
               sems, *, m, nt):
    j = pl.program_id(1)
    n_src = feat_hbm.shape[0]
    chunk = min(_CHUNK, n_src)
    nchunks = n_src // chunk

    @pl.when(j == 0)
    def _load_interleaved():
        tmps = (tmp0, tmp1)

        def cp(c):
            return pltpu.make_async_copy(
                feat_hbm.at[pl.ds(c * chunk, chunk), :],
                tmps[c % 2], sems.at[c % 2])

        for c in range(min(2, nchunks)):
            cp(c).start()
        for c in range(nchunks):
            cp(c).wait()
            if c + 2 < nchunks:
                cp(c + 2).start()
            tc = tmps[c % 2]

            def body(r, _, c=c, tc=tc):
                rb = pl.multiple_of(r * 32, 8)
                b = c * (2 * chunk) + r * 64
                for u in range(4):
                    v = tc[pl.ds(rb + 8 * u, 8), :]
                    buf0 = b + 16 * u
                    fbuf[pl.Slice(buf0, 8, 2), :] = v[:, 0:128]
                    fbuf[pl.Slice(buf0 + 1, 8, 2), :] = v[:, 128:256]
                return 0

            lax.fori_loop(0, chunk // 32, body, 0)

    t = pl.program_id(0) * nt + j
    x = _gather_sum_tile(idx_ref, fbuf, buf, t * (m * _FANOUT), m, 2)
    y = jnp.dot(x, w_ref[...], preferred_element_type=jnp.float32) + b_ref[...]
    h = jnp.maximum(y, 0.0)
    o_ref[0:2 * m:2, :] = h[:, :128]
    o_ref[1:2 * m:2, :] = h[:, 128:]


def _l1_kernel(idx_ref, src_ref, w1_ref, wa_ref, wb_ref, b1_ref, o_ref, buf,
               *, m, nt):
    t = pl.program_id(0) * nt + pl.program_id(1)
    x = _gather_sum_tile(idx_ref, src_ref, buf, t * (m * _FANOUT), m, 2)
    y = jnp.dot(x, w1_ref[...], preferred_element_type=jnp.float32) + b1_ref[...]
    yr = jnp.maximum(y, 0.0)
    o_ref[...] = (jnp.dot(y, wa_ref[...], preferred_element_type=jnp.float32)
                  + jnp.dot(yr, wb_ref[...], preferred_element_type=jnp.float32))


def _l2_kernel(idx_ref, src_ref, b2_ref, o_ref, *, m, nt):
    t = pl.program_id(0) * nt + pl.program_id(1)
    base = t * (m * _FANOUT)
    bias = b2_ref[0]
    for mi in range(m):
        o = base + _FANOUT * mi
        acc = (src_ref[idx_ref[o], 0] + src_ref[idx_ref[o + 1], 0]
               + src_ref[idx_ref[o + 2], 0] + src_ref[idx_ref[o + 3], 0])
        o_ref[mi] = acc + bias


def _layer0(feat, idx, w, b, *, m):
    n_dst = idx.shape[0] // _FANOUT
    fin, fout = w.shape
    nt = n_dst // (2 * m)
    kern = functools.partial(_l0_kernel, m=m, nt=nt)
    return pl.pallas_call(
        kern,
        out_shape=jax.ShapeDtypeStruct((n_dst * (fout // 128), 128),
                                       jnp.float32),
        grid_spec=pltpu.PrefetchScalarGridSpec(
            num_scalar_prefetch=1,
            grid=(2, nt),
            in_specs=[
                pl.BlockSpec(memory_space=pl.ANY),
                pl.BlockSpec((fin, fout), lambda i, j, idx: (0, 0)),
                pl.BlockSpec((1, fout), lambda i, j, idx: (0, 0)),
            ],
            out_specs=pl.BlockSpec((m * (fout // 128), 128),
                                   lambda i, j, idx: (i * nt + j, 0)),
            scratch_shapes=[
                pltpu.VMEM((feat.shape[0] * 2, 128), jnp.float32),
                pltpu.VMEM((min(_CHUNK, feat.shape[0]), 256), jnp.float32),
                pltpu.VMEM((min(_CHUNK, feat.shape[0]), 256), jnp.float32),
                pltpu.VMEM((2 * (m + 1), 128), jnp.float32),
                pltpu.SemaphoreType.DMA((2,)),
            ],
        ),
        compiler_params=pltpu.CompilerParams(
            dimension_semantics=("parallel", "arbitrary"),
            vmem_limit_bytes=48 << 20,
        ),
    )(idx, feat, w, b)


def _layer1(src2, idx, w1, wa, wb, b1, *, m):
    n_dst = idx.shape[0] // _FANOUT
    fin, fmid = w1.shape
    fout = wa.shape[1]
    nt = n_dst // (2 * m)
    kern = functools.partial(_l1_kernel, m=m, nt=nt)
    return pl.pallas_call(
        kern,
        out_shape=jax.ShapeDtypeStruct((n_dst, fout), jnp.float32),
        grid_spec=pltpu.PrefetchScalarGridSpec(
            num_scalar_prefetch=1,
            grid=(2, nt),
            in_specs=[
                pl.BlockSpec(src2.shape, lambda i, j, idx: (0, 0)),
                pl.BlockSpec((fin, fmid), lambda i, j, idx: (0, 0)),
                pl.BlockSpec((fmid, fout), lambda i, j, idx: (0, 0)),
                pl.BlockSpec((fmid, fout), lambda i, j, idx: (0, 0)),
                pl.BlockSpec((1, fmid), lambda i, j, idx: (0, 0)),
            ],
            out_specs=pl.BlockSpec((m, fout), lambda i, j, idx: (i * nt + j, 0)),
            scratch_shapes=[
                pltpu.VMEM((2 * (m + 1), 128), jnp.float32),
            ],
        ),
        compiler_params=pltpu.CompilerParams(
            dimension_semantics=("parallel", "arbitrary"),
            vmem_limit_bytes=40 << 20,
        ),
    )(idx, src2, w1, wa, wb, b1)


def _layer2(src3, idx, b2, *, m):
    n_dst = idx.shape[0] // _FANOUT
    fout = src3.shape[-1]
    nt = n_dst // (2 * m)
    kern = functools.partial(_l2_kernel, m=m, nt=nt)
    return pl.pallas_call(
        kern,
        out_shape=jax.ShapeDtypeStruct((n_dst, fout), jnp.float32),
        grid_spec=pltpu.PrefetchScalarGridSpec(
            num_scalar_prefetch=1,
            grid=(2, nt),
            in_specs=[
                pl.BlockSpec(src3.shape, lambda i, j, idx: (0, 0, 0)),
                pl.BlockSpec((1, fout), lambda i, j, idx: (0, 0)),
            ],
            out_specs=pl.BlockSpec((m, fout), lambda i, j, idx: (i * nt + j, 0)),
            scratch_shapes=[],
        ),
        compiler_params=pltpu.CompilerParams(
            dimension_semantics=("parallel", "arbitrary"),
            vmem_limit_bytes=16 << 20,
        ),
    )(idx, src3, b2)


def kernel(features, w0, b0, w1, b1, w2, b2, nbr0, nbr1, nbr2):
    f32 = jnp.float32
    fin = features.shape[1]
    fmid = w1.shape[0]

    idx0 = (nbr0.astype(jnp.int32) * (fin // 128)).reshape(-1)
    w0s = (w0.astype(f32) / _FANOUT)
    h1v = _layer0(features.astype(f32), idx0, w0s,
                  b0.astype(f32).reshape(1, -1), m=256)

    idx1 = (nbr1.astype(jnp.int32) * (fmid // 128)).reshape(-1)
    w1s = (w1.astype(f32) / _FANOUT)
    wa = (w2[:fmid].astype(f32) / _FANOUT)
    wb = (w2[fmid:].astype(f32) / _FANOUT)
    z = _layer1(h1v, idx1, w1s, wa, wb, b1.astype(f32).reshape(1, -1), m=256)

    idx2 = nbr2.astype(jnp.int32).reshape(-1)
    out = _layer2(z.reshape(z.shape[0], 1, z.shape[1]), idx2,
                  b2.astype(f32).reshape(1, -1), m=256)
    return out.astype(f32)


# speedup vs baseline: 1.3852x; 1.0022x over previous
import functools

import jax
import jax.numpy as jnp
from jax import lax
from jax.experimental import pallas as pl
from jax.experimental.pallas import tpu as pltpu

_FANOUT = 4
_CHUNK = 8192


def _gather_sum_tile(idx_ref, src, buf, base, m, p):
    S = m + 1
    for mi in range(m):
        o = base + _FANOUT * mi
        acc = None
        for k in range(_FANOUT):
            ik = pl.multiple_of(idx_ref[o + k], p)
            slab = src[pl.ds(ik, p), :]
            acc = slab if acc is None else acc + slab
        buf[mi:mi + p * S:S, :] = acc
    return jnp.concatenate([buf[c * S:c * S + m, :] for c in range(p)],
                           axis=-1)


def _l0_kernel(idx_ref, feat_hbm, w_ref, b_ref, o_ref, fbuf, tmp0, tmp1, buf,
               sems, *, m, nt):
    j = pl.program_id(1)
    n_src = feat_hbm.shape[0]
    chunk = min(_CHUNK, n_src)
    nchunks = n_src // chunk

    @pl.when(j == 0)
    def _load_interleaved():
        tmps = (tmp0, tmp1)

        def cp(c):
            return pltpu.make_async_copy(
                feat_hbm.at[pl.ds(c * chunk, chunk), :],
                tmps[c % 2], sems.at[c % 2])

        for c in range(min(2, nchunks)):
            cp(c).start()
        for c in range(nchunks):
            cp(c).wait()
            if c + 2 < nchunks:
                cp(c + 2).start()
            tc = tmps[c % 2]

            def body(r, _, c=c, tc=tc):
                rb = pl.multiple_of(r * 32, 8)
                b = c * (2 * chunk) + r * 64
                for u in range(4):
                    v = tc[pl.ds(rb + 8 * u, 8), :]
                    buf0 = b + 16 * u
                    fbuf[pl.Slice(buf0, 8, 2), :] = v[:, 0:128]
                    fbuf[pl.Slice(buf0 + 1, 8, 2), :] = v[:, 128:256]
                return 0

            lax.fori_loop(0, chunk // 32, body, 0)

    t = pl.program_id(0) * nt + j
    x = _gather_sum_tile(idx_ref, fbuf, buf, t * (m * _FANOUT), m, 2)
    y = jnp.dot(x, w_ref[...], preferred_element_type=jnp.float32) + b_ref[...]
    h = jnp.maximum(y, 0.0)
    o_ref[0:2 * m:2, :] = h[:, :128]
    o_ref[1:2 * m:2, :] = h[:, 128:]


def _l1_kernel(idx_ref, src_ref, w1_ref, wa_ref, wb_ref, b1_ref, o_ref, buf,
               *, m, nt):
    t = pl.program_id(0) * nt + pl.program_id(1)
    x = _gather_sum_tile(idx_ref, src_ref, buf, t * (m * _FANOUT), m, 2)
    y = jnp.dot(x, w1_ref[...], preferred_element_type=jnp.float32) + b1_ref[...]
    yr = jnp.maximum(y, 0.0)
    o_ref[...] = (jnp.dot(y, wa_ref[...], preferred_element_type=jnp.float32)
                  + jnp.dot(yr, wb_ref[...], preferred_element_type=jnp.float32))


def _l2_kernel(idx_ref, src_ref, b2_ref, o_ref, *, m, nt):
    t = pl.program_id(0) * nt + pl.program_id(1)
    base = t * (m * _FANOUT)
    bias = b2_ref[...]
    for mi in range(m):
        o = base + _FANOUT * mi
        acc = (src_ref[pl.ds(idx_ref[o], 1), :] + src_ref[pl.ds(idx_ref[o + 1], 1), :]
               + src_ref[pl.ds(idx_ref[o + 2], 1), :] + src_ref[pl.ds(idx_ref[o + 3], 1), :])
        o_ref[pl.ds(mi, 1), :] = acc + bias


def _layer0(feat, idx, w, b, *, m):
    n_dst = idx.shape[0] // _FANOUT
    fin, fout = w.shape
    nt = n_dst // (2 * m)
    kern = functools.partial(_l0_kernel, m=m, nt=nt)
    return pl.pallas_call(
        kern,
        out_shape=jax.ShapeDtypeStruct((n_dst * (fout // 128), 128),
                                       jnp.float32),
        grid_spec=pltpu.PrefetchScalarGridSpec(
            num_scalar_prefetch=1,
            grid=(2, nt),
            in_specs=[
                pl.BlockSpec(memory_space=pl.ANY),
                pl.BlockSpec((fin, fout), lambda i, j, idx: (0, 0)),
                pl.BlockSpec((1, fout), lambda i, j, idx: (0, 0)),
            ],
            out_specs=pl.BlockSpec((m * (fout // 128), 128),
                                   lambda i, j, idx: (i * nt + j, 0)),
            scratch_shapes=[
                pltpu.VMEM((feat.shape[0] * 2, 128), jnp.float32),
                pltpu.VMEM((min(_CHUNK, feat.shape[0]), 256), jnp.float32),
                pltpu.VMEM((min(_CHUNK, feat.shape[0]), 256), jnp.float32),
                pltpu.VMEM((2 * (m + 1), 128), jnp.float32),
                pltpu.SemaphoreType.DMA((2,)),
            ],
        ),
        compiler_params=pltpu.CompilerParams(
            dimension_semantics=("parallel", "arbitrary"),
            vmem_limit_bytes=56 << 20,
        ),
    )(idx, feat, w, b)


def _layer1(src2, idx, w1, wa, wb, b1, *, m):
    n_dst = idx.shape[0] // _FANOUT
    fin, fmid = w1.shape
    fout = wa.shape[1]
    nt = n_dst // (2 * m)
    kern = functools.partial(_l1_kernel, m=m, nt=nt)
    return pl.pallas_call(
        kern,
        out_shape=jax.ShapeDtypeStruct((n_dst, fout), jnp.float32),
        grid_spec=pltpu.PrefetchScalarGridSpec(
            num_scalar_prefetch=1,
            grid=(2, nt),
            in_specs=[
                pl.BlockSpec(src2.shape, lambda i, j, idx: (0, 0)),
                pl.BlockSpec((fin, fmid), lambda i, j, idx: (0, 0)),
                pl.BlockSpec((fmid, fout), lambda i, j, idx: (0, 0)),
                pl.BlockSpec((fmid, fout), lambda i, j, idx: (0, 0)),
                pl.BlockSpec((1, fmid), lambda i, j, idx: (0, 0)),
            ],
            out_specs=pl.BlockSpec((m, fout), lambda i, j, idx: (i * nt + j, 0)),
            scratch_shapes=[
                pltpu.VMEM((2 * (m + 1), 128), jnp.float32),
            ],
        ),
        compiler_params=pltpu.CompilerParams(
            dimension_semantics=("parallel", "arbitrary"),
            vmem_limit_bytes=40 << 20,
        ),
    )(idx, src2, w1, wa, wb, b1)


def _layer2(src3, idx, b2, *, m):
    n_dst = idx.shape[0] // _FANOUT
    fout = src3.shape[-1]
    nt = n_dst // (2 * m)
    kern = functools.partial(_l2_kernel, m=m, nt=nt)
    return pl.pallas_call(
        kern,
        out_shape=jax.ShapeDtypeStruct((n_dst, fout), jnp.float32),
        grid_spec=pltpu.PrefetchScalarGridSpec(
            num_scalar_prefetch=1,
            grid=(2, nt),
            in_specs=[
                pl.BlockSpec(src3.shape, lambda i, j, idx: (0, 0)),
                pl.BlockSpec((1, fout), lambda i, j, idx: (0, 0)),
            ],
            out_specs=pl.BlockSpec((m, fout), lambda i, j, idx: (i * nt + j, 0)),
            scratch_shapes=[],
        ),
        compiler_params=pltpu.CompilerParams(
            dimension_semantics=("parallel", "arbitrary"),
            vmem_limit_bytes=16 << 20,
        ),
    )(idx, src3, b2)


def kernel(features, w0, b0, w1, b1, w2, b2, nbr0, nbr1, nbr2):
    f32 = jnp.float32
    fin = features.shape[1]
    fmid = w1.shape[0]

    idx0 = (nbr0.astype(jnp.int32) * (fin // 128)).reshape(-1)
    w0s = (w0.astype(f32) / _FANOUT)
    h1v = _layer0(features.astype(f32), idx0, w0s,
                  b0.astype(f32).reshape(1, -1), m=256)

    idx1 = (nbr1.astype(jnp.int32) * (fmid // 128)).reshape(-1)
    w1s = (w1.astype(f32) / _FANOUT)
    wa = (w2[:fmid].astype(f32) / _FANOUT)
    wb = (w2[fmid:].astype(f32) / _FANOUT)
    z = _layer1(h1v, idx1, w1s, wa, wb, b1.astype(f32).reshape(1, -1), m=256)

    idx2 = nbr2.astype(jnp.int32).reshape(-1)
    out = _layer2(z, idx2, b2.astype(f32).reshape(1, -1), m=256)
    return out.astype(f32)


# speedup vs baseline: 1.3995x; 1.0104x over previous
import functools

import jax
import jax.numpy as jnp
from jax import lax
from jax.experimental import pallas as pl
from jax.experimental.pallas import tpu as pltpu

_FANOUT = 4
_CHUNK = 8192


def _gather_sum_tile(idx_ref, src, buf, base, m, p):
    S = m + 1
    for mi in range(m):
        o = base + _FANOUT * mi
        acc = None
        for k in range(_FANOUT):
            ik = pl.multiple_of(idx_ref[o + k], p)
            slab = src[pl.ds(ik, p), :]
            acc = slab if acc is None else acc + slab
        buf[mi:mi + p * S:S, :] = acc
    return jnp.concatenate([buf[c * S:c * S + m, :] for c in range(p)],
                           axis=-1)


def _l0_kernel(idx_ref, feat_hbm, w_ref, b_ref, o_ref, fbuf, tmp0, tmp1, buf,
               sems, *, m, nt):
    j = pl.program_id(1)
    n_src = feat_hbm.shape[0]
    chunk = min(_CHUNK, n_src)
    nchunks = n_src // chunk

    @pl.when(j == 0)
    def _load_interleaved():
        tmps = (tmp0, tmp1)

        def cp(c):
            return pltpu.make_async_copy(
                feat_hbm.at[pl.ds(c * chunk, chunk), :],
                tmps[c % 2], sems.at[c % 2])

        for c in range(min(2, nchunks)):
            cp(c).start()
        for c in range(nchunks):
            cp(c).wait()
            if c + 2 < nchunks:
                cp(c + 2).start()
            tc = tmps[c % 2]

            def body(r, _, c=c, tc=tc):
                rb = pl.multiple_of(r * 32, 8)
                b = c * (2 * chunk) + r * 64
                for u in range(4):
                    v = tc[pl.ds(rb + 8 * u, 8), :]
                    buf0 = b + 16 * u
                    fbuf[pl.Slice(buf0, 8, 2), :] = v[:, 0:128]
                    fbuf[pl.Slice(buf0 + 1, 8, 2), :] = v[:, 128:256]
                return 0

            lax.fori_loop(0, chunk // 32, body, 0)

    t = pl.program_id(0) * nt + j
    x = _gather_sum_tile(idx_ref, fbuf, buf, t * (m * _FANOUT), m, 2)
    y = (jnp.dot(x.astype(jnp.bfloat16), w_ref[...],
                 preferred_element_type=jnp.float32) + b_ref[...])
    h = jnp.maximum(y, 0.0)
    o_ref[0:2 * m:2, :] = h[:, :128]
    o_ref[1:2 * m:2, :] = h[:, 128:]


def _l1_kernel(idx_ref, src_ref, w1_ref, wa_ref, wb_ref, b1_ref, o_ref, buf,
               *, m, nt):
    t = pl.program_id(0) * nt + pl.program_id(1)
    x = _gather_sum_tile(idx_ref, src_ref, buf, t * (m * _FANOUT), m, 2)
    y = (jnp.dot(x.astype(jnp.bfloat16), w1_ref[...],
                 preferred_element_type=jnp.float32) + b1_ref[...])
    yr = jnp.maximum(y, 0.0)
    o_ref[...] = (jnp.dot(y.astype(jnp.bfloat16), wa_ref[...],
                          preferred_element_type=jnp.float32)
                  + jnp.dot(yr.astype(jnp.bfloat16), wb_ref[...],
                            preferred_element_type=jnp.float32))


def _l2_kernel(idx_ref, src_ref, b2_ref, o_ref, *, m, nt):
    t = pl.program_id(0) * nt + pl.program_id(1)
    base = t * (m * _FANOUT)
    bias = b2_ref[...]
    for mi in range(m):
        o = base + _FANOUT * mi
        acc = (src_ref[pl.ds(idx_ref[o], 1), :] + src_ref[pl.ds(idx_ref[o + 1], 1), :]
               + src_ref[pl.ds(idx_ref[o + 2], 1), :] + src_ref[pl.ds(idx_ref[o + 3], 1), :])
        o_ref[pl.ds(mi, 1), :] = acc + bias


def _layer0(feat, idx, w, b, *, m):
    n_dst = idx.shape[0] // _FANOUT
    fin, fout = w.shape
    nt = n_dst // (2 * m)
    kern = functools.partial(_l0_kernel, m=m, nt=nt)
    return pl.pallas_call(
        kern,
        out_shape=jax.ShapeDtypeStruct((n_dst * (fout // 128), 128),
                                       jnp.float32),
        grid_spec=pltpu.PrefetchScalarGridSpec(
            num_scalar_prefetch=1,
            grid=(2, nt),
            in_specs=[
                pl.BlockSpec(memory_space=pl.ANY),
                pl.BlockSpec((fin, fout), lambda i, j, idx: (0, 0)),
                pl.BlockSpec((1, fout), lambda i, j, idx: (0, 0)),
            ],
            out_specs=pl.BlockSpec((m * (fout // 128), 128),
                                   lambda i, j, idx: (i * nt + j, 0)),
            scratch_shapes=[
                pltpu.VMEM((feat.shape[0] * 2, 128), jnp.float32),
                pltpu.VMEM((min(_CHUNK, feat.shape[0]), 256), jnp.float32),
                pltpu.VMEM((min(_CHUNK, feat.shape[0]), 256), jnp.float32),
                pltpu.VMEM((2 * (m + 1), 128), jnp.float32),
                pltpu.SemaphoreType.DMA((2,)),
            ],
        ),
        compiler_params=pltpu.CompilerParams(
            dimension_semantics=("parallel", "arbitrary"),
            vmem_limit_bytes=56 << 20,
        ),
    )(idx, feat, w, b)


def _layer1(src2, idx, w1, wa, wb, b1, *, m):
    n_dst = idx.shape[0] // _FANOUT
    fin, fmid = w1.shape
    fout = wa.shape[1]
    nt = n_dst // (2 * m)
    kern = functools.partial(_l1_kernel, m=m, nt=nt)
    return pl.pallas_call(
        kern,
        out_shape=jax.ShapeDtypeStruct((n_dst, fout), jnp.float32),
        grid_spec=pltpu.PrefetchScalarGridSpec(
            num_scalar_prefetch=1,
            grid=(2, nt),
            in_specs=[
                pl.BlockSpec(src2.shape, lambda i, j, idx: (0, 0)),
                pl.BlockSpec((fin, fmid), lambda i, j, idx: (0, 0)),
                pl.BlockSpec((fmid, fout), lambda i, j, idx: (0, 0)),
                pl.BlockSpec((fmid, fout), lambda i, j, idx: (0, 0)),
                pl.BlockSpec((1, fmid), lambda i, j, idx: (0, 0)),
            ],
            out_specs=pl.BlockSpec((m, fout), lambda i, j, idx: (i * nt + j, 0)),
            scratch_shapes=[
                pltpu.VMEM((2 * (m + 1), 128), jnp.float32),
            ],
        ),
        compiler_params=pltpu.CompilerParams(
            dimension_semantics=("parallel", "arbitrary"),
            vmem_limit_bytes=40 << 20,
        ),
    )(idx, src2, w1, wa, wb, b1)


def _layer2(src3, idx, b2, *, m):
    n_dst = idx.shape[0] // _FANOUT
    fout = src3.shape[-1]
    nt = n_dst // (2 * m)
    kern = functools.partial(_l2_kernel, m=m, nt=nt)
    return pl.pallas_call(
        kern,
        out_shape=jax.ShapeDtypeStruct((n_dst, fout), jnp.float32),
        grid_spec=pltpu.PrefetchScalarGridSpec(
            num_scalar_prefetch=1,
            grid=(2, nt),
            in_specs=[
                pl.BlockSpec(src3.shape, lambda i, j, idx: (0, 0)),
                pl.BlockSpec((1, fout), lambda i, j, idx: (0, 0)),
            ],
            out_specs=pl.BlockSpec((m, fout), lambda i, j, idx: (i * nt + j, 0)),
            scratch_shapes=[],
        ),
        compiler_params=pltpu.CompilerParams(
            dimension_semantics=("parallel", "arbitrary"),
            vmem_limit_bytes=16 << 20,
        ),
    )(idx, src3, b2)


def kernel(features, w0, b0, w1, b1, w2, b2, nbr0, nbr1, nbr2):
    f32 = jnp.float32
    fin = features.shape[1]
    fmid = w1.shape[0]

    idx0 = (nbr0.astype(jnp.int32) * (fin // 128)).reshape(-1)
    w0s = (w0.astype(f32) / _FANOUT).astype(jnp.bfloat16)
    h1v = _layer0(features.astype(f32), idx0, w0s,
                  b0.astype(f32).reshape(1, -1), m=256)

    idx1 = (nbr1.astype(jnp.int32) * (fmid // 128)).reshape(-1)
    w1s = (w1.astype(f32) / _FANOUT).astype(jnp.bfloat16)
    wa = (w2[:fmid].astype(f32) / _FANOUT).astype(jnp.bfloat16)
    wb = (w2[fmid:].astype(f32) / _FANOUT).astype(jnp.bfloat16)
    z = _layer1(h1v, idx1, w1s, wa, wb, b1.astype(f32).reshape(1, -1), m=256)

    idx2 = nbr2.astype(jnp.int32).reshape(-1)
    out = _layer2(z, idx2, b2.astype(f32).reshape(1, -1), m=256)
    return out.astype(f32)


# speedup vs baseline: 1.4285x; 1.0207x over previous
import functools

import jax
import jax.numpy as jnp
from jax import lax
from jax.experimental import pallas as pl
from jax.experimental.pallas import tpu as pltpu

_FANOUT = 4
_CHUNK = 4096
_NBUF = 4


def _gather_sum_tile(idx_ref, src, buf, base, m, p):
    S = m + 1
    for mi in range(m):
        o = base + _FANOUT * mi
        acc = None
        for k in range(_FANOUT):
            ik = pl.multiple_of(idx_ref[o + k], p)
            slab = src[pl.ds(ik, p), :]
            acc = slab if acc is None else acc + slab
        buf[mi:mi + p * S:S, :] = acc
    return jnp.concatenate([buf[c * S:c * S + m, :] for c in range(p)],
                           axis=-1)


def _l0_kernel(idx_ref, feat_hbm, w_ref, b_ref, o_ref, fbuf, tmp0, tmp1, tmp2,
               tmp3, buf, sems, *, m, nt):
    j = pl.program_id(1)
    n_src = feat_hbm.shape[0]
    chunk = min(_CHUNK, n_src)
    nchunks = n_src // chunk

    @pl.when(j == 0)
    def _load_interleaved():
        tmps = (tmp0, tmp1, tmp2, tmp3)

        def cp(c):
            return pltpu.make_async_copy(
                feat_hbm.at[pl.ds(c * chunk, chunk), :],
                tmps[c % _NBUF], sems.at[c % _NBUF])

        for c in range(min(_NBUF, nchunks)):
            cp(c).start()
        for c in range(nchunks):
            cp(c).wait()
            if c + _NBUF < nchunks:
                cp(c + _NBUF).start()
            tc = tmps[c % _NBUF]

            def body(r, _, c=c, tc=tc):
                rb = pl.multiple_of(r * 32, 8)
                b = c * (2 * chunk) + r * 64
                for u in range(4):
                    v = tc[pl.ds(rb + 8 * u, 8), :]
                    buf0 = b + 16 * u
                    fbuf[pl.Slice(buf0, 8, 2), :] = v[:, 0:128]
                    fbuf[pl.Slice(buf0 + 1, 8, 2), :] = v[:, 128:256]
                return 0

            lax.fori_loop(0, chunk // 32, body, 0)

    t = pl.program_id(0) * nt + j
    x = _gather_sum_tile(idx_ref, fbuf, buf, t * (m * _FANOUT), m, 2)
    y = (jnp.dot(x.astype(jnp.bfloat16), w_ref[...],
                 preferred_element_type=jnp.float32) + b_ref[...])
    h = jnp.maximum(y, 0.0)
    o_ref[0:2 * m:2, :] = h[:, :128]
    o_ref[1:2 * m:2, :] = h[:, 128:]


def _l1_kernel(idx_ref, src_ref, w1_ref, wa_ref, wb_ref, b1_ref, o_ref, buf,
               *, m, nt):
    t = pl.program_id(0) * nt + pl.program_id(1)
    x = _gather_sum_tile(idx_ref, src_ref, buf, t * (m * _FANOUT), m, 2)
    y = (jnp.dot(x.astype(jnp.bfloat16), w1_ref[...],
                 preferred_element_type=jnp.float32) + b1_ref[...])
    yr = jnp.maximum(y, 0.0)
    o_ref[...] = (jnp.dot(y.astype(jnp.bfloat16), wa_ref[...],
                          preferred_element_type=jnp.float32)
                  + jnp.dot(yr.astype(jnp.bfloat16), wb_ref[...],
                            preferred_element_type=jnp.float32))


def _l2_kernel(idx_ref, src_ref, b2_ref, o_ref, *, m, nt):
    t = pl.program_id(0) * nt + pl.program_id(1)
    base = t * (m * _FANOUT)
    bias = b2_ref[...]
    for mi in range(m):
        o = base + _FANOUT * mi
        acc = (src_ref[pl.ds(idx_ref[o], 1), :] + src_ref[pl.ds(idx_ref[o + 1], 1), :]
               + src_ref[pl.ds(idx_ref[o + 2], 1), :] + src_ref[pl.ds(idx_ref[o + 3], 1), :])
        o_ref[pl.ds(mi, 1), :] = acc + bias


def _layer0(feat, idx, w, b, *, m):
    n_dst = idx.shape[0] // _FANOUT
    fin, fout = w.shape
    nt = n_dst // (2 * m)
    kern = functools.partial(_l0_kernel, m=m, nt=nt)
    return pl.pallas_call(
        kern,
        out_shape=jax.ShapeDtypeStruct((n_dst * (fout // 128), 128),
                                       jnp.float32),
        grid_spec=pltpu.PrefetchScalarGridSpec(
            num_scalar_prefetch=1,
            grid=(2, nt),
            in_specs=[
                pl.BlockSpec(memory_space=pl.ANY),
                pl.BlockSpec((fin, fout), lambda i, j, idx: (0, 0)),
                pl.BlockSpec((1, fout), lambda i, j, idx: (0, 0)),
            ],
            out_specs=pl.BlockSpec((m * (fout // 128), 128),
                                   lambda i, j, idx: (i * nt + j, 0)),
            scratch_shapes=[
                pltpu.VMEM((feat.shape[0] * 2, 128), jnp.float32),
                pltpu.VMEM((min(_CHUNK, feat.shape[0]), 256), jnp.float32),
                pltpu.VMEM((min(_CHUNK, feat.shape[0]), 256), jnp.float32),
                pltpu.VMEM((min(_CHUNK, feat.shape[0]), 256), jnp.float32),
                pltpu.VMEM((min(_CHUNK, feat.shape[0]), 256), jnp.float32),
                pltpu.VMEM((2 * (m + 1), 128), jnp.float32),
                pltpu.SemaphoreType.DMA((4,)),
            ],
        ),
        compiler_params=pltpu.CompilerParams(
            dimension_semantics=("parallel", "arbitrary"),
            vmem_limit_bytes=56 << 20,
        ),
    )(idx, feat, w, b)


def _layer1(src2, idx, w1, wa, wb, b1, *, m):
    n_dst = idx.shape[0] // _FANOUT
    fin, fmid = w1.shape
    fout = wa.shape[1]
    nt = n_dst // (2 * m)
    kern = functools.partial(_l1_kernel, m=m, nt=nt)
    return pl.pallas_call(
        kern,
        out_shape=jax.ShapeDtypeStruct((n_dst, fout), jnp.float32),
        grid_spec=pltpu.PrefetchScalarGridSpec(
            num_scalar_prefetch=1,
            grid=(2, nt),
            in_specs=[
                pl.BlockSpec(src2.shape, lambda i, j, idx: (0, 0)),
                pl.BlockSpec((fin, fmid), lambda i, j, idx: (0, 0)),
                pl.BlockSpec((fmid, fout), lambda i, j, idx: (0, 0)),
                pl.BlockSpec((fmid, fout), lambda i, j, idx: (0, 0)),
                pl.BlockSpec((1, fmid), lambda i, j, idx: (0, 0)),
            ],
            out_specs=pl.BlockSpec((m, fout), lambda i, j, idx: (i * nt + j, 0)),
            scratch_shapes=[
                pltpu.VMEM((2 * (m + 1), 128), jnp.float32),
            ],
        ),
        compiler_params=pltpu.CompilerParams(
            dimension_semantics=("parallel", "arbitrary"),
            vmem_limit_bytes=40 << 20,
        ),
    )(idx, src2, w1, wa, wb, b1)


def _layer2(src3, idx, b2, *, m):
    n_dst = idx.shape[0] // _FANOUT
    fout = src3.shape[-1]
    nt = n_dst // (2 * m)
    kern = functools.partial(_l2_kernel, m=m, nt=nt)
    return pl.pallas_call(
        kern,
        out_shape=jax.ShapeDtypeStruct((n_dst, fout), jnp.float32),
        grid_spec=pltpu.PrefetchScalarGridSpec(
            num_scalar_prefetch=1,
            grid=(2, nt),
            in_specs=[
                pl.BlockSpec(src3.shape, lambda i, j, idx: (0, 0)),
                pl.BlockSpec((1, fout), lambda i, j, idx: (0, 0)),
            ],
            out_specs=pl.BlockSpec((m, fout), lambda i, j, idx: (i * nt + j, 0)),
            scratch_shapes=[],
        ),
        compiler_params=pltpu.CompilerParams(
            dimension_semantics=("parallel", "arbitrary"),
            vmem_limit_bytes=16 << 20,
        ),
    )(idx, src3, b2)


def kernel(features, w0, b0, w1, b1, w2, b2, nbr0, nbr1, nbr2):
    f32 = jnp.float32
    fin = features.shape[1]
    fmid = w1.shape[0]

    idx0 = (nbr0.astype(jnp.int32) * (fin // 128)).reshape(-1)
    w0s = (w0.astype(f32) / _FANOUT).astype(jnp.bfloat16)
    h1v = _layer0(features.astype(f32), idx0, w0s,
                  b0.astype(f32).reshape(1, -1), m=256)

    idx1 = (nbr1.astype(jnp.int32) * (fmid // 128)).reshape(-1)
    w1s = (w1.astype(f32) / _FANOUT).astype(jnp.bfloat16)
    wa = (w2[:fmid].astype(f32) / _FANOUT).astype(jnp.bfloat16)
    wb = (w2[fmid:].astype(f32) / _FANOUT).astype(jnp.bfloat16)
    z = _layer1(h1v, idx1, w1s, wa, wb, b1.astype(f32).reshape(1, -1), m=256)

    idx2 = nbr2.astype(jnp.int32).reshape(-1)
    out = _layer2(z, idx2, b2.astype(f32).reshape(1, -1), m=256)
    return out.astype(f32)


# speedup vs baseline: 1.6197x; 1.1338x over previous
import functools

import jax
import jax.numpy as jnp
from jax import lax
from jax.experimental import pallas as pl
from jax.experimental.pallas import tpu as pltpu

_FANOUT = 4
_CHUNK = 4096


def _gather_sum_tile(idx_ref, src, buf, base, m):
    S = m + 1
    for mi in range(m):
        o = base + _FANOUT * mi
        acc = None
        for k in range(_FANOUT):
            ik = pl.multiple_of(idx_ref[o + k], 2)
            slab = src[pl.ds(ik, 2), :]
            acc = slab if acc is None else acc + slab
        buf[mi:mi + 2 * S:S, :] = acc
    return jnp.concatenate([buf[0:m, :], buf[S:S + m, :]], axis=-1)


def _gcn_kernel(idx0_ref, idx1_ref, idx2_ref, feat_hbm,
                w0_ref, b0_ref, w1_ref, wa_ref, wb_ref, b1_ref, b2_ref,
                o_ref, fbuf, h1v, zbuf, tmp0, tmp1, buf, sems,
                *, m, nt0, nt1, nt2):
    j = pl.program_id(0)
    n_src = feat_hbm.shape[0]
    chunk = min(_CHUNK, n_src)
    nchunks = n_src // chunk
    bf16 = jnp.bfloat16
    f32 = jnp.float32

    @pl.when(j == 0)
    def _load_interleaved():
        tmps = (tmp0, tmp1)

        def cp(c):
            return pltpu.make_async_copy(
                feat_hbm.at[pl.ds(c * chunk, chunk), :],
                tmps[c % 2], sems.at[c % 2])

        for c in range(min(2, nchunks)):
            cp(c).start()
        for c in range(nchunks):
            cp(c).wait()
            if c + 2 < nchunks:
                cp(c + 2).start()
            tc = tmps[c % 2]

            def body(r, _, c=c, tc=tc):
                rb = pl.multiple_of(r * 32, 8)
                b = c * (2 * chunk) + r * 64
                for u in range(4):
                    v = tc[pl.ds(rb + 8 * u, 8), :]
                    b0_ = b + 16 * u
                    fbuf[pl.Slice(b0_, 8, 2), :] = v[:, 0:128]
                    fbuf[pl.Slice(b0_ + 1, 8, 2), :] = v[:, 128:256]
                return 0

            lax.fori_loop(0, chunk // 32, body, 0)

    @pl.when(j < nt0)
    def _l0():
        x = _gather_sum_tile(idx0_ref, fbuf, buf, j * (m * _FANOUT), m)
        y = (jnp.dot(x.astype(bf16), w0_ref[...],
                     preferred_element_type=f32) + b0_ref[...])
        h = jnp.maximum(y, 0.0)
        h1v[pl.Slice(2 * m * j, m, 2), :] = h[:, :128]
        h1v[pl.Slice(2 * m * j + 1, m, 2), :] = h[:, 128:]

    @pl.when((j >= nt0) & (j < nt0 + nt1))
    def _l1():
        t = j - nt0
        x = _gather_sum_tile(idx1_ref, h1v, buf, t * (m * _FANOUT), m)
        y = (jnp.dot(x.astype(bf16), w1_ref[...],
                     preferred_element_type=f32) + b1_ref[...])
        yr = jnp.maximum(y, 0.0)
        z = (jnp.dot(y.astype(bf16), wa_ref[...], preferred_element_type=f32)
             + jnp.dot(yr.astype(bf16), wb_ref[...],
                       preferred_element_type=f32))
        zbuf[pl.ds(t * m, m), :] = z

    @pl.when(j >= nt0 + nt1)
    def _l2():
        t = j - (nt0 + nt1)
        base = t * (m * _FANOUT)
        bias = b2_ref[...]
        for mi in range(m):
            o = base + _FANOUT * mi
            acc = (zbuf[pl.ds(idx2_ref[o], 1), :]
                   + zbuf[pl.ds(idx2_ref[o + 1], 1), :]
                   + zbuf[pl.ds(idx2_ref[o + 2], 1), :]
                   + zbuf[pl.ds(idx2_ref[o + 3], 1), :])
            o_ref[pl.ds(mi, 1), :] = acc + bias


def _gcn_call(feat, idx0, idx1, idx2, w0, b0, w1, wa, wb, b1, b2, *, m):
    n_src, fin = feat.shape
    n1 = idx0.shape[0] // _FANOUT
    n2 = idx1.shape[0] // _FANOUT
    n3 = idx2.shape[0] // _FANOUT
    fout = wa.shape[1]
    nt0, nt1, nt2 = n1 // m, n2 // m, n3 // m
    chunk = min(_CHUNK, n_src)
    kern = functools.partial(_gcn_kernel, m=m, nt0=nt0, nt1=nt1, nt2=nt2)
    s = nt0 + nt1
    return pl.pallas_call(
        kern,
        out_shape=jax.ShapeDtypeStruct((n3, fout), jnp.float32),
        grid_spec=pltpu.PrefetchScalarGridSpec(
            num_scalar_prefetch=3,
            grid=(nt0 + nt1 + nt2,),
            in_specs=[
                pl.BlockSpec(memory_space=pl.ANY),
                pl.BlockSpec(w0.shape, lambda j, i0, i1, i2: (0, 0)),
                pl.BlockSpec(b0.shape, lambda j, i0, i1, i2: (0, 0)),
                pl.BlockSpec(w1.shape, lambda j, i0, i1, i2: (0, 0)),
                pl.BlockSpec(wa.shape, lambda j, i0, i1, i2: (0, 0)),
                pl.BlockSpec(wb.shape, lambda j, i0, i1, i2: (0, 0)),
                pl.BlockSpec(b1.shape, lambda j, i0, i1, i2: (0, 0)),
                pl.BlockSpec(b2.shape, lambda j, i0, i1, i2: (0, 0)),
            ],
            out_specs=pl.BlockSpec(
                (m, fout),
                lambda j, i0, i1, i2: (jnp.maximum(j - s, 0), 0)),
            scratch_shapes=[
                pltpu.VMEM((2 * n_src, 128), jnp.float32),
                pltpu.VMEM((2 * n1, 128), jnp.float32),
                pltpu.VMEM((n2, fout), jnp.float32),
                pltpu.VMEM((chunk, fin), jnp.float32),
                pltpu.VMEM((chunk, fin), jnp.float32),
                pltpu.VMEM((2 * (m + 1), 128), jnp.float32),
                pltpu.SemaphoreType.DMA((2,)),
            ],
        ),
        compiler_params=pltpu.CompilerParams(
            dimension_semantics=("arbitrary",),
            vmem_limit_bytes=56 << 20,
        ),
    )(idx0, idx1, idx2, feat, w0, b0, w1, wa, wb, b1, b2)


def kernel(features, w0, b0, w1, b1, w2, b2, nbr0, nbr1, nbr2):
    f32 = jnp.float32
    bf16 = jnp.bfloat16
    fmid = w1.shape[0]

    idx0 = (nbr0.astype(jnp.int32) * 2).reshape(-1)
    idx1 = (nbr1.astype(jnp.int32) * 2).reshape(-1)
    idx2 = nbr2.astype(jnp.int32).reshape(-1)
    w0s = (w0.astype(f32) / _FANOUT).astype(bf16)
    w1s = (w1.astype(f32) / _FANOUT).astype(bf16)
    wa = (w2[:fmid].astype(f32) / _FANOUT).astype(bf16)
    wb = (w2[fmid:].astype(f32) / _FANOUT).astype(bf16)
    out = _gcn_call(features.astype(f32), idx0, idx1, idx2,
                    w0s, b0.astype(f32).reshape(1, -1),
                    w1s, wa, wb,
                    b1.astype(f32).reshape(1, -1),
                    b2.astype(f32).reshape(1, -1), m=256)
    return out.astype(f32)


# speedup vs baseline: 1.6308x; 1.0069x over previous
import functools

import jax
import jax.numpy as jnp
from jax import lax
from jax.experimental import pallas as pl
from jax.experimental.pallas import tpu as pltpu

_FANOUT = 4
_CHUNK = 2048
_NBUF = 4


def _gather_sum_tile(idx_ref, src, buf, base, m):
    S = m + 1
    for mi in range(m):
        o = base + _FANOUT * mi
        acc = None
        for k in range(_FANOUT):
            ik = pl.multiple_of(idx_ref[o + k], 2)
            slab = src[pl.ds(ik, 2), :]
            acc = slab if acc is None else acc + slab
        buf[mi:mi + 2 * S:S, :] = acc
    return jnp.concatenate([buf[0:m, :], buf[S:S + m, :]], axis=-1)


def _gcn_kernel(idx0_ref, idx1_ref, idx2_ref, feat_hbm,
                w0_ref, b0_ref, w1_ref, wa_ref, wb_ref, b1_ref, b2_ref,
                o_ref, fbuf, h1v, zbuf, tmp0, tmp1, tmp2, tmp3, buf, sems,
                *, m, nt0, nt1, nt2):
    j = pl.program_id(0)
    n_src = feat_hbm.shape[0]
    chunk = min(_CHUNK, n_src)
    nchunks = n_src // chunk
    bf16 = jnp.bfloat16
    f32 = jnp.float32

    @pl.when(j == 0)
    def _load_interleaved():
        tmps = (tmp0, tmp1, tmp2, tmp3)

        def cp(c):
            return pltpu.make_async_copy(
                feat_hbm.at[pl.ds(c * chunk, chunk), :],
                tmps[c % _NBUF], sems.at[c % _NBUF])

        for c in range(min(_NBUF, nchunks)):
            cp(c).start()
        for c in range(nchunks):
            cp(c).wait()
            if c + _NBUF < nchunks:
                cp(c + _NBUF).start()
            tc = tmps[c % _NBUF]

            def body(r, _, c=c, tc=tc):
                rb = pl.multiple_of(r * 32, 8)
                b = c * (2 * chunk) + r * 64
                for u in range(4):
                    v = tc[pl.ds(rb + 8 * u, 8), :]
                    b0_ = b + 16 * u
                    fbuf[pl.Slice(b0_, 8, 2), :] = v[:, 0:128]
                    fbuf[pl.Slice(b0_ + 1, 8, 2), :] = v[:, 128:256]
                return 0

            lax.fori_loop(0, chunk // 32, body, 0)

    @pl.when(j < nt0)
    def _l0():
        x = _gather_sum_tile(idx0_ref, fbuf, buf, j * (m * _FANOUT), m)
        y = (jnp.dot(x.astype(bf16), w0_ref[...],
                     preferred_element_type=f32) + b0_ref[...])
        h = jnp.maximum(y, 0.0)
        h1v[pl.Slice(2 * m * j, m, 2), :] = h[:, :128]
        h1v[pl.Slice(2 * m * j + 1, m, 2), :] = h[:, 128:]

    @pl.when((j >= nt0) & (j < nt0 + nt1))
    def _l1():
        t = j - nt0
        x = _gather_sum_tile(idx1_ref, h1v, buf, t * (m * _FANOUT), m)
        y = (jnp.dot(x.astype(bf16), w1_ref[...],
                     preferred_element_type=f32) + b1_ref[...])
        yr = jnp.maximum(y, 0.0)
        z = (jnp.dot(y.astype(bf16), wa_ref[...], preferred_element_type=f32)
             + jnp.dot(yr.astype(bf16), wb_ref[...],
                       preferred_element_type=f32))
        zbuf[pl.ds(t * m, m), :] = z

    @pl.when(j >= nt0 + nt1)
    def _l2():
        t = j - (nt0 + nt1)
        base = t * (m * _FANOUT)
        bias = b2_ref[...]
        for mi in range(m):
            o = base + _FANOUT * mi
            acc = (zbuf[pl.ds(idx2_ref[o], 1), :]
                   + zbuf[pl.ds(idx2_ref[o + 1], 1), :]
                   + zbuf[pl.ds(idx2_ref[o + 2], 1), :]
                   + zbuf[pl.ds(idx2_ref[o + 3], 1), :])
            o_ref[pl.ds(mi, 1), :] = acc + bias


def _gcn_call(feat, idx0, idx1, idx2, w0, b0, w1, wa, wb, b1, b2, *, m):
    n_src, fin = feat.shape
    n1 = idx0.shape[0] // _FANOUT
    n2 = idx1.shape[0] // _FANOUT
    n3 = idx2.shape[0] // _FANOUT
    fout = wa.shape[1]
    nt0, nt1, nt2 = n1 // m, n2 // m, n3 // m
    chunk = min(_CHUNK, n_src)
    kern = functools.partial(_gcn_kernel, m=m, nt0=nt0, nt1=nt1, nt2=nt2)
    s = nt0 + nt1
    return pl.pallas_call(
        kern,
        out_shape=jax.ShapeDtypeStruct((n3, fout), jnp.float32),
        grid_spec=pltpu.PrefetchScalarGridSpec(
            num_scalar_prefetch=3,
            grid=(nt0 + nt1 + nt2,),
            in_specs=[
                pl.BlockSpec(memory_space=pl.ANY),
                pl.BlockSpec(w0.shape, lambda j, i0, i1, i2: (0, 0)),
                pl.BlockSpec(b0.shape, lambda j, i0, i1, i2: (0, 0)),
                pl.BlockSpec(w1.shape, lambda j, i0, i1, i2: (0, 0)),
                pl.BlockSpec(wa.shape, lambda j, i0, i1, i2: (0, 0)),
                pl.BlockSpec(wb.shape, lambda j, i0, i1, i2: (0, 0)),
                pl.BlockSpec(b1.shape, lambda j, i0, i1, i2: (0, 0)),
                pl.BlockSpec(b2.shape, lambda j, i0, i1, i2: (0, 0)),
            ],
            out_specs=pl.BlockSpec(
                (m, fout),
                lambda j, i0, i1, i2: (jnp.maximum(j - s, 0), 0)),
            scratch_shapes=[
                pltpu.VMEM((2 * n_src, 128), jnp.float32),
                pltpu.VMEM((2 * n1, 128), jnp.float32),
                pltpu.VMEM((n2, fout), jnp.float32),
                pltpu.VMEM((chunk, fin), jnp.float32),
                pltpu.VMEM((chunk, fin), jnp.float32),
                pltpu.VMEM((chunk, fin), jnp.float32),
                pltpu.VMEM((chunk, fin), jnp.float32),
                pltpu.VMEM((2 * (m + 1), 128), jnp.float32),
                pltpu.SemaphoreType.DMA((4,)),
            ],
        ),
        compiler_params=pltpu.CompilerParams(
            dimension_semantics=("arbitrary",),
            vmem_limit_bytes=56 << 20,
        ),
    )(idx0, idx1, idx2, feat, w0, b0, w1, wa, wb, b1, b2)


def kernel(features, w0, b0, w1, b1, w2, b2, nbr0, nbr1, nbr2):
    f32 = jnp.float32
    bf16 = jnp.bfloat16
    fmid = w1.shape[0]

    idx0 = (nbr0.astype(jnp.int32) * 2).reshape(-1)
    idx1 = (nbr1.astype(jnp.int32) * 2).reshape(-1)
    idx2 = nbr2.astype(jnp.int32).reshape(-1)
    w0s = (w0.astype(f32) / _FANOUT).astype(bf16)
    w1s = (w1.astype(f32) / _FANOUT).astype(bf16)
    wa = (w2[:fmid].astype(f32) / _FANOUT).astype(bf16)
    wb = (w2[fmid:].astype(f32) / _FANOUT).astype(bf16)
    out = _gcn_call(features.astype(f32), idx0, idx1, idx2,
                    w0s, b0.astype(f32).reshape(1, -1),
                    w1s, wa, wb,
                    b1.astype(f32).reshape(1, -1),
                    b2.astype(f32).reshape(1, -1), m=256)
    return out.astype(f32)


# speedup vs baseline: 1.7095x; 1.0483x over previous
import functools

import jax
import jax.numpy as jnp
from jax import lax
from jax.experimental import pallas as pl
from jax.experimental.pallas import tpu as pltpu

_FANOUT = 4
_CHUNK = 2048
_NBUF = 4


def _gather_sum_tile(idx_ref, src, buf, base, m):
    S = m + 1
    for mi in range(m):
        o = base + _FANOUT * mi
        acc = None
        for k in range(_FANOUT):
            ik = pl.multiple_of(idx_ref[o + k], 2)
            slab = src[pl.ds(ik, 2), :]
            acc = slab if acc is None else acc + slab
        buf[mi:mi + 2 * S:S, :] = acc
    return jnp.concatenate([buf[0:m, :], buf[S:S + m, :]], axis=-1)


def _gcn_kernel(idx0_ref, idx1_ref, idx2_ref, feat_hbm,
                w0_ref, b0_ref, w1_ref, wa_ref, wb_ref, b1_ref, b2_ref,
                o_ref, fbuf, h1v, zbuf, tmp0, tmp1, tmp2, tmp3, buf, sems,
                *, m, nt0, nt1, nt2):
    j = pl.program_id(0)
    n_src = feat_hbm.shape[0]
    chunk = min(_CHUNK, n_src)
    nchunks = n_src // chunk
    bf16 = jnp.bfloat16
    f32 = jnp.float32

    @pl.when(j == 0)
    def _load_interleaved():
        tmps = (tmp0, tmp1, tmp2, tmp3)

        def cp(c):
            return pltpu.make_async_copy(
                feat_hbm.at[pl.ds(c * chunk, chunk), :],
                tmps[c % _NBUF], sems.at[c % _NBUF])

        for c in range(min(_NBUF, nchunks)):
            cp(c).start()
        for c in range(nchunks):
            cp(c).wait()
            if c + _NBUF < nchunks:
                cp(c + _NBUF).start()
            tc = tmps[c % _NBUF]

            def body(r, _, c=c, tc=tc):
                rb = pl.multiple_of(r * 32, 8)
                b = c * (2 * chunk) + r * 64
                for u in range(4):
                    v = tc[pl.ds(rb + 8 * u, 8), :]
                    b0_ = b + 16 * u
                    fbuf[pl.Slice(b0_, 8, 2), :] = v[:, 0:128]
                    fbuf[pl.Slice(b0_ + 1, 8, 2), :] = v[:, 128:256]
                return 0

            lax.fori_loop(0, chunk // 32, body, 0)

    @pl.when(j < nt0)
    def _l0():
        x = _gather_sum_tile(idx0_ref, fbuf, buf, j * (m * _FANOUT), m)
        y = (jnp.dot(x.astype(bf16), w0_ref[...],
                     preferred_element_type=f32) + b0_ref[...])
        h = jnp.maximum(y, 0.0)
        h1v[pl.Slice(2 * m * j, m, 2), :] = h[:, :128]
        h1v[pl.Slice(2 * m * j + 1, m, 2), :] = h[:, 128:]

    @pl.when((j >= nt0) & (j < nt0 + nt1))
    def _l1():
        t = j - nt0
        x = _gather_sum_tile(idx1_ref, h1v, buf, t * (m * _FANOUT), m)
        y = (jnp.dot(x.astype(bf16), w1_ref[...],
                     preferred_element_type=f32) + b1_ref[...])
        yr = jnp.maximum(y, 0.0)
        z = (jnp.dot(y.astype(bf16), wa_ref[...], preferred_element_type=f32)
             + jnp.dot(yr.astype(bf16), wb_ref[...],
                       preferred_element_type=f32))
        zbuf[pl.ds(t * m, m), :] = z

    @pl.when(j >= nt0 + nt1)
    def _l2():
        t = j - (nt0 + nt1)
        base = t * (m * _FANOUT)
        bias = b2_ref[...]
        for mi in range(m):
            o = base + _FANOUT * mi
            acc = (zbuf[pl.ds(idx2_ref[o], 1), :]
                   + zbuf[pl.ds(idx2_ref[o + 1], 1), :]
                   + zbuf[pl.ds(idx2_ref[o + 2], 1), :]
                   + zbuf[pl.ds(idx2_ref[o + 3], 1), :])
            o_ref[pl.ds(mi, 1), :] = acc + bias


def _gcn_call(feat, idx0, idx1, idx2, w0, b0, w1, wa, wb, b1, b2, *, m):
    n_src, fin = feat.shape
    n1 = idx0.shape[0] // _FANOUT
    n2 = idx1.shape[0] // _FANOUT
    n3 = idx2.shape[0] // _FANOUT
    fout = wa.shape[1]
    nt0, nt1, nt2 = n1 // m, n2 // m, n3 // m
    chunk = min(_CHUNK, n_src)
    kern = functools.partial(_gcn_kernel, m=m, nt0=nt0, nt1=nt1, nt2=nt2)
    s = nt0 + nt1
    return pl.pallas_call(
        kern,
        out_shape=jax.ShapeDtypeStruct((n3, fout), jnp.float32),
        grid_spec=pltpu.PrefetchScalarGridSpec(
            num_scalar_prefetch=3,
            grid=(nt0 + nt1 + nt2,),
            in_specs=[
                pl.BlockSpec(memory_space=pl.ANY),
                pl.BlockSpec(w0.shape, lambda j, i0, i1, i2: (0, 0)),
                pl.BlockSpec(b0.shape, lambda j, i0, i1, i2: (0, 0)),
                pl.BlockSpec(w1.shape, lambda j, i0, i1, i2: (0, 0)),
                pl.BlockSpec(wa.shape, lambda j, i0, i1, i2: (0, 0)),
                pl.BlockSpec(wb.shape, lambda j, i0, i1, i2: (0, 0)),
                pl.BlockSpec(b1.shape, lambda j, i0, i1, i2: (0, 0)),
                pl.BlockSpec(b2.shape, lambda j, i0, i1, i2: (0, 0)),
            ],
            out_specs=pl.BlockSpec(
                (m, fout),
                lambda j, i0, i1, i2: (jnp.maximum(j - s, 0), 0)),
            scratch_shapes=[
                pltpu.VMEM((2 * n_src, 128), jnp.float32),
                pltpu.VMEM((2 * n1, 128), jnp.float32),
                pltpu.VMEM((n2, fout), jnp.float32),
                pltpu.VMEM((chunk, fin), jnp.float32),
                pltpu.VMEM((chunk, fin), jnp.float32),
                pltpu.VMEM((chunk, fin), jnp.float32),
                pltpu.VMEM((chunk, fin), jnp.float32),
                pltpu.VMEM((2 * (m + 1), 128), jnp.float32),
                pltpu.SemaphoreType.DMA((4,)),
            ],
        ),
        compiler_params=pltpu.CompilerParams(
            dimension_semantics=("arbitrary",),
            vmem_limit_bytes=56 << 20,
        ),
    )(idx0, idx1, idx2, feat, w0, b0, w1, wa, wb, b1, b2)


def kernel(features, w0, b0, w1, b1, w2, b2, nbr0, nbr1, nbr2):
    f32 = jnp.float32
    bf16 = jnp.bfloat16
    fmid = w1.shape[0]

    idx0 = (nbr0.astype(jnp.int32) * 2).reshape(-1)
    idx1 = (nbr1.astype(jnp.int32) * 2).reshape(-1)
    idx2 = nbr2.astype(jnp.int32).reshape(-1)
    w0s = (w0.astype(f32) / _FANOUT).astype(bf16)
    w1s = (w1.astype(f32) / _FANOUT).astype(bf16)
    wa = (w2[:fmid].astype(f32) / _FANOUT).astype(bf16)
    wb = (w2[fmid:].astype(f32) / _FANOUT).astype(bf16)
    out = _gcn_call(features.astype(f32), idx0, idx1, idx2,
                    w0s, b0.astype(f32).reshape(1, -1),
                    w1s, wa, wb,
                    b1.astype(f32).reshape(1, -1),
                    b2.astype(f32).reshape(1, -1), m=512)
    return out.astype(f32)


# speedup vs baseline: 1.7651x; 1.0325x over previous
import functools

import jax
import jax.numpy as jnp
from jax import lax
from jax.experimental import pallas as pl
from jax.experimental.pallas import tpu as pltpu

_FANOUT = 4
_CHUNK = 2048
_NBUF = 4


def _gather_sum_tile(idx_ref, src, buf, base, m):
    S = m + 1
    for mi in range(m):
        o = base + _FANOUT * mi
        acc = None
        for k in range(_FANOUT):
            ik = pl.multiple_of(idx_ref[o + k], 2)
            slab = src[pl.ds(ik, 2), :]
            acc = slab if acc is None else acc + slab
        buf[mi:mi + 2 * S:S, :] = acc
    return jnp.concatenate([buf[0:m, :], buf[S:S + m, :]], axis=-1)


def _gcn_kernel(idx0_ref, idx1_ref, idx2_ref, feat_hbm,
                w0_ref, b0_ref, w1_ref, wa_ref, wb_ref, b1_ref, b2_ref,
                o_ref, fbuf, h1v, zbuf, tmp0, tmp1, tmp2, tmp3, buf, sems,
                *, m, nt0, nt1, nt2):
    j = pl.program_id(0)
    n_src = feat_hbm.shape[0]
    chunk = min(_CHUNK, n_src)
    nchunks = n_src // chunk
    bf16 = jnp.bfloat16
    f32 = jnp.float32

    @pl.when(j == 0)
    def _load_interleaved():
        tmps = (tmp0, tmp1, tmp2, tmp3)

        def cp(c):
            return pltpu.make_async_copy(
                feat_hbm.at[pl.ds(c * chunk, chunk), :],
                tmps[c % _NBUF], sems.at[c % _NBUF])

        for c in range(min(_NBUF, nchunks)):
            cp(c).start()
        for c in range(nchunks):
            cp(c).wait()
            if c + _NBUF < nchunks:
                cp(c + _NBUF).start()
            tc = tmps[c % _NBUF]

            def body(r, _, c=c, tc=tc):
                rb = pl.multiple_of(r * 32, 8)
                b = c * (2 * chunk) + r * 64
                for u in range(4):
                    v = tc[pl.ds(rb + 8 * u, 8), :]
                    b0_ = b + 16 * u
                    fbuf[pl.Slice(b0_, 8, 2), :] = v[:, 0:128]
                    fbuf[pl.Slice(b0_ + 1, 8, 2), :] = v[:, 128:256]
                return 0

            lax.fori_loop(0, chunk // 32, body, 0)

    @pl.when(j < nt0)
    def _l0():
        x = _gather_sum_tile(idx0_ref, fbuf, buf, j * (m * _FANOUT), m)
        y = (jnp.dot(x.astype(bf16), w0_ref[...],
                     preferred_element_type=f32) + b0_ref[...])
        h = jnp.maximum(y, 0.0)
        h1v[pl.Slice(2 * m * j, m, 2), :] = h[:, :128]
        h1v[pl.Slice(2 * m * j + 1, m, 2), :] = h[:, 128:]

    @pl.when((j >= nt0) & (j < nt0 + nt1))
    def _l1():
        t = j - nt0
        x = _gather_sum_tile(idx1_ref, h1v, buf, t * (m * _FANOUT), m)
        y = (jnp.dot(x.astype(bf16), w1_ref[...],
                     preferred_element_type=f32) + b1_ref[...])
        yr = jnp.maximum(y, 0.0)
        z = (jnp.dot(y.astype(bf16), wa_ref[...], preferred_element_type=f32)
             + jnp.dot(yr.astype(bf16), wb_ref[...],
                       preferred_element_type=f32))
        zbuf[pl.ds(t * m, m), :] = z

    @pl.when(j >= nt0 + nt1)
    def _l2():
        t = j - (nt0 + nt1)
        base = t * (m * _FANOUT)
        bias = b2_ref[...]
        for mi in range(m):
            o = base + _FANOUT * mi
            acc = (zbuf[pl.ds(idx2_ref[o], 1), :]
                   + zbuf[pl.ds(idx2_ref[o + 1], 1), :]
                   + zbuf[pl.ds(idx2_ref[o + 2], 1), :]
                   + zbuf[pl.ds(idx2_ref[o + 3], 1), :])
            o_ref[pl.ds(mi, 1), :] = acc + bias


def _gcn_call(feat, idx0, idx1, idx2, w0, b0, w1, wa, wb, b1, b2, *, m):
    n_src, fin = feat.shape
    n1 = idx0.shape[0] // _FANOUT
    n2 = idx1.shape[0] // _FANOUT
    n3 = idx2.shape[0] // _FANOUT
    fout = wa.shape[1]
    nt0, nt1, nt2 = n1 // m, n2 // m, n3 // m
    chunk = min(_CHUNK, n_src)
    kern = functools.partial(_gcn_kernel, m=m, nt0=nt0, nt1=nt1, nt2=nt2)
    s = nt0 + nt1
    return pl.pallas_call(
        kern,
        out_shape=jax.ShapeDtypeStruct((n3, fout), jnp.float32),
        grid_spec=pltpu.PrefetchScalarGridSpec(
            num_scalar_prefetch=3,
            grid=(nt0 + nt1 + nt2,),
            in_specs=[
                pl.BlockSpec(memory_space=pl.ANY),
                pl.BlockSpec(w0.shape, lambda j, i0, i1, i2: (0, 0)),
                pl.BlockSpec(b0.shape, lambda j, i0, i1, i2: (0, 0)),
                pl.BlockSpec(w1.shape, lambda j, i0, i1, i2: (0, 0)),
                pl.BlockSpec(wa.shape, lambda j, i0, i1, i2: (0, 0)),
                pl.BlockSpec(wb.shape, lambda j, i0, i1, i2: (0, 0)),
                pl.BlockSpec(b1.shape, lambda j, i0, i1, i2: (0, 0)),
                pl.BlockSpec(b2.shape, lambda j, i0, i1, i2: (0, 0)),
            ],
            out_specs=pl.BlockSpec(
                (m, fout),
                lambda j, i0, i1, i2: (jnp.maximum(j - s, 0), 0)),
            scratch_shapes=[
                pltpu.VMEM((2 * n_src, 128), jnp.float32),
                pltpu.VMEM((2 * n1, 128), jnp.float32),
                pltpu.VMEM((n2, fout), jnp.float32),
                pltpu.VMEM((chunk, fin), jnp.float32),
                pltpu.VMEM((chunk, fin), jnp.float32),
                pltpu.VMEM((chunk, fin), jnp.float32),
                pltpu.VMEM((chunk, fin), jnp.float32),
                pltpu.VMEM((2 * (m + 1), 128), jnp.float32),
                pltpu.SemaphoreType.DMA((4,)),
            ],
        ),
        compiler_params=pltpu.CompilerParams(
            dimension_semantics=("arbitrary",),
            vmem_limit_bytes=56 << 20,
        ),
    )(idx0, idx1, idx2, feat, w0, b0, w1, wa, wb, b1, b2)


def kernel(features, w0, b0, w1, b1, w2, b2, nbr0, nbr1, nbr2):
    f32 = jnp.float32
    bf16 = jnp.bfloat16
    fmid = w1.shape[0]

    idx0 = (nbr0.astype(jnp.int32) * 2).reshape(-1)
    idx1 = (nbr1.astype(jnp.int32) * 2).reshape(-1)
    idx2 = nbr2.astype(jnp.int32).reshape(-1)
    w0s = (w0.astype(f32) / _FANOUT).astype(bf16)
    w1s = (w1.astype(f32) / _FANOUT).astype(bf16)
    wa = (w2[:fmid].astype(f32) / _FANOUT).astype(bf16)
    wb = (w2[fmid:].astype(f32) / _FANOUT).astype(bf16)
    out = _gcn_call(features.astype(f32), idx0, idx1, idx2,
                    w0s, b0.astype(f32).reshape(1, -1),
                    w1s, wa, wb,
                    b1.astype(f32).reshape(1, -1),
                    b2.astype(f32).reshape(1, -1), m=1024)
    return out.astype(f32)


# speedup vs baseline: 1.7735x; 1.0047x over previous
import functools

import jax
import jax.numpy as jnp
from jax import lax
from jax.experimental import pallas as pl
from jax.experimental.pallas import tpu as pltpu

_FANOUT = 4
_CHUNK = 2048
_NBUF = 4


def _gather_sum_tile(idx_ref, src, buf, base, m):
    S = m + 1
    for mi in range(m):
        o = base + _FANOUT * mi
        acc = None
        for k in range(_FANOUT):
            ik = pl.multiple_of(idx_ref[o + k], 2)
            slab = src[pl.ds(ik, 2), :]
            acc = slab if acc is None else acc + slab
        buf[mi:mi + 2 * S:S, :] = acc
    return jnp.concatenate([buf[0:m, :], buf[S:S + m, :]], axis=-1)


def _gcn_kernel(idx0_ref, idx1_ref, idx2_ref, feat_hbm,
                w0_ref, b0_ref, w1_ref, wa_ref, wb_ref, b1_ref, b2_ref,
                o_ref, fbuf, h1v, zbuf, tmp0, tmp1, tmp2, tmp3, buf, buf2, sems,
                *, m, nt0, nt1, nt2):
    j = pl.program_id(0)
    n_src = feat_hbm.shape[0]
    chunk = min(_CHUNK, n_src)
    nchunks = n_src // chunk
    bf16 = jnp.bfloat16
    f32 = jnp.float32

    @pl.when(j == 0)
    def _load_interleaved():
        tmps = (tmp0, tmp1, tmp2, tmp3)

        def cp(c):
            return pltpu.make_async_copy(
                feat_hbm.at[pl.ds(c * chunk, chunk), :],
                tmps[c % _NBUF], sems.at[c % _NBUF])

        for c in range(min(_NBUF, nchunks)):
            cp(c).start()
        for c in range(nchunks):
            cp(c).wait()
            if c + _NBUF < nchunks:
                cp(c + _NBUF).start()
            tc = tmps[c % _NBUF]

            def body(r, _, c=c, tc=tc):
                rb = pl.multiple_of(r * 32, 8)
                b = c * (2 * chunk) + r * 64
                for u in range(4):
                    v = tc[pl.ds(rb + 8 * u, 8), :]
                    b0_ = b + 16 * u
                    fbuf[pl.Slice(b0_, 8, 2), :] = v[:, 0:128]
                    fbuf[pl.Slice(b0_ + 1, 8, 2), :] = v[:, 128:256]
                return 0

            lax.fori_loop(0, chunk // 32, body, 0)

    @pl.when(j < nt0)
    def _l0():
        hm = m // 2
        for half, bh in ((0, buf), (1, buf2)):
            base = (2 * j + half) * (hm * _FANOUT)
            x = _gather_sum_tile(idx0_ref, fbuf, bh, base, hm)
            y = (jnp.dot(x.astype(bf16), w0_ref[...],
                         preferred_element_type=f32) + b0_ref[...])
            h = jnp.maximum(y, 0.0)
            r0 = 2 * m * j + 2 * hm * half
            h1v[pl.Slice(r0, hm, 2), :] = h[:, :128]
            h1v[pl.Slice(r0 + 1, hm, 2), :] = h[:, 128:]

    @pl.when((j >= nt0) & (j < nt0 + nt1))
    def _l1():
        t = j - nt0
        hm = m // 2
        for half, bh in ((0, buf), (1, buf2)):
            base = (2 * t + half) * (hm * _FANOUT)
            x = _gather_sum_tile(idx1_ref, h1v, bh, base, hm)
            y = (jnp.dot(x.astype(bf16), w1_ref[...],
                         preferred_element_type=f32) + b1_ref[...])
            yr = jnp.maximum(y, 0.0)
            z = (jnp.dot(y.astype(bf16), wa_ref[...],
                         preferred_element_type=f32)
                 + jnp.dot(yr.astype(bf16), wb_ref[...],
                           preferred_element_type=f32))
            zbuf[pl.ds(t * m + hm * half, hm), :] = z

    @pl.when(j >= nt0 + nt1)
    def _l2():
        t = j - (nt0 + nt1)
        base = t * (m * _FANOUT)
        bias = b2_ref[...]
        for mi in range(m):
            o = base + _FANOUT * mi
            acc = (zbuf[pl.ds(idx2_ref[o], 1), :]
                   + zbuf[pl.ds(idx2_ref[o + 1], 1), :]
                   + zbuf[pl.ds(idx2_ref[o + 2], 1), :]
                   + zbuf[pl.ds(idx2_ref[o + 3], 1), :])
            o_ref[pl.ds(mi, 1), :] = acc + bias


def _gcn_call(feat, idx0, idx1, idx2, w0, b0, w1, wa, wb, b1, b2, *, m):
    n_src, fin = feat.shape
    n1 = idx0.shape[0] // _FANOUT
    n2 = idx1.shape[0] // _FANOUT
    n3 = idx2.shape[0] // _FANOUT
    fout = wa.shape[1]
    nt0, nt1, nt2 = n1 // m, n2 // m, n3 // m
    chunk = min(_CHUNK, n_src)
    kern = functools.partial(_gcn_kernel, m=m, nt0=nt0, nt1=nt1, nt2=nt2)
    s = nt0 + nt1
    return pl.pallas_call(
        kern,
        out_shape=jax.ShapeDtypeStruct((n3, fout), jnp.float32),
        grid_spec=pltpu.PrefetchScalarGridSpec(
            num_scalar_prefetch=3,
            grid=(nt0 + nt1 + nt2,),
            in_specs=[
                pl.BlockSpec(memory_space=pl.ANY),
                pl.BlockSpec(w0.shape, lambda j, i0, i1, i2: (0, 0)),
                pl.BlockSpec(b0.shape, lambda j, i0, i1, i2: (0, 0)),
                pl.BlockSpec(w1.shape, lambda j, i0, i1, i2: (0, 0)),
                pl.BlockSpec(wa.shape, lambda j, i0, i1, i2: (0, 0)),
                pl.BlockSpec(wb.shape, lambda j, i0, i1, i2: (0, 0)),
                pl.BlockSpec(b1.shape, lambda j, i0, i1, i2: (0, 0)),
                pl.BlockSpec(b2.shape, lambda j, i0, i1, i2: (0, 0)),
            ],
            out_specs=pl.BlockSpec(
                (m, fout),
                lambda j, i0, i1, i2: (jnp.maximum(j - s, 0), 0)),
            scratch_shapes=[
                pltpu.VMEM((2 * n_src, 128), jnp.float32),
                pltpu.VMEM((2 * n1, 128), jnp.float32),
                pltpu.VMEM((n2, fout), jnp.float32),
                pltpu.VMEM((chunk, fin), jnp.float32),
                pltpu.VMEM((chunk, fin), jnp.float32),
                pltpu.VMEM((chunk, fin), jnp.float32),
                pltpu.VMEM((chunk, fin), jnp.float32),
                pltpu.VMEM((m + 2, 128), jnp.float32),
                pltpu.VMEM((m + 2, 128), jnp.float32),
                pltpu.SemaphoreType.DMA((4,)),
            ],
        ),
        compiler_params=pltpu.CompilerParams(
            dimension_semantics=("arbitrary",),
            vmem_limit_bytes=56 << 20,
        ),
    )(idx0, idx1, idx2, feat, w0, b0, w1, wa, wb, b1, b2)


def kernel(features, w0, b0, w1, b1, w2, b2, nbr0, nbr1, nbr2):
    f32 = jnp.float32
    bf16 = jnp.bfloat16
    fmid = w1.shape[0]

    idx0 = (nbr0.astype(jnp.int32) * 2).reshape(-1)
    idx1 = (nbr1.astype(jnp.int32) * 2).reshape(-1)
    idx2 = nbr2.astype(jnp.int32).reshape(-1)
    w0s = (w0.astype(f32) / _FANOUT).astype(bf16)
    w1s = (w1.astype(f32) / _FANOUT).astype(bf16)
    wa = (w2[:fmid].astype(f32) / _FANOUT).astype(bf16)
    wb = (w2[fmid:].astype(f32) / _FANOUT).astype(bf16)
    out = _gcn_call(features.astype(f32), idx0, idx1, idx2,
                    w0s, b0.astype(f32).reshape(1, -1),
                    w1s, wa, wb,
                    b1.astype(f32).reshape(1, -1),
                    b2.astype(f32).reshape(1, -1), m=1024)
    return out.astype(f32)


# speedup vs baseline: 1.8005x; 1.0153x over previous
import functools

import jax
import jax.numpy as jnp
from jax import lax
from jax.experimental import pallas as pl
from jax.experimental.pallas import tpu as pltpu

_FANOUT = 4
_CHUNK = 2048
_NBUF = 4


def _gather_sum_tile(idx_ref, src, buf, base, m):
    S = m + 1
    for mi in range(m):
        o = base + _FANOUT * mi
        acc = None
        for k in range(_FANOUT):
            ik = pl.multiple_of(idx_ref[o + k], 2)
            slab = src[pl.ds(ik, 2), :]
            acc = slab if acc is None else acc + slab
        buf[mi:mi + 2 * S:S, :] = acc
    return jnp.concatenate([buf[0:m, :], buf[S:S + m, :]], axis=-1)


def _gcn_kernel(idx0_ref, idx1_ref, idx2_ref, feat_hbm,
                w0_ref, b0_ref, w1_ref, wa_ref, wb_ref, b1_ref, b2_ref,
                o_ref, fbuf, h1v, zbuf, tmp0, tmp1, tmp2, tmp3, buf, buf2, sems,
                *, m, nt0, nt1, nt2):
    j = pl.program_id(0)
    n_src = feat_hbm.shape[0]
    chunk = min(_CHUNK, n_src)
    nchunks = n_src // chunk
    bf16 = jnp.bfloat16
    f32 = jnp.float32

    @pl.when(j == 0)
    def _load_interleaved():
        tmps = (tmp0, tmp1, tmp2, tmp3)

        def cp(c):
            return pltpu.make_async_copy(
                feat_hbm.at[pl.ds(c * chunk, chunk), :],
                tmps[c % _NBUF], sems.at[c % _NBUF])

        for c in range(min(_NBUF, nchunks)):
            cp(c).start()
        for c in range(nchunks):
            cp(c).wait()
            if c + _NBUF < nchunks:
                cp(c + _NBUF).start()
            tc = tmps[c % _NBUF]

            def body(r, _, c=c, tc=tc):
                rb = pl.multiple_of(r * 32, 8)
                b = c * (2 * chunk) + r * 64
                for u in range(4):
                    v = tc[pl.ds(rb + 8 * u, 8), :]
                    b0_ = b + 16 * u
                    fbuf[pl.Slice(b0_, 8, 2), :] = v[:, 0:128]
                    fbuf[pl.Slice(b0_ + 1, 8, 2), :] = v[:, 128:256]
                return 0

            lax.fori_loop(0, chunk // 32, body, 0)

    @pl.when(j < nt0)
    def _l0():
        hm = m // 2
        for half, bh in ((0, buf), (1, buf2)):
            base = (2 * j + half) * (hm * _FANOUT)
            x = _gather_sum_tile(idx0_ref, fbuf, bh, base, hm)
            y = (jnp.dot(x.astype(bf16), w0_ref[...],
                         preferred_element_type=f32) + b0_ref[...])
            h = jnp.maximum(y, 0.0)
            r0 = 2 * m * j + 2 * hm * half
            h1v[pl.Slice(r0, hm, 2), :] = h[:, :128]
            h1v[pl.Slice(r0 + 1, hm, 2), :] = h[:, 128:]

    @pl.when((j >= nt0) & (j < nt0 + nt1))
    def _l1():
        t = j - nt0
        hm = m // 2
        for half, bh in ((0, buf), (1, buf2)):
            base = (2 * t + half) * (hm * _FANOUT)
            x = _gather_sum_tile(idx1_ref, h1v, bh, base, hm)
            y = (jnp.dot(x.astype(bf16), w1_ref[...],
                         preferred_element_type=f32) + b1_ref[...])
            yr = jnp.maximum(y, 0.0)
            z = (jnp.dot(y.astype(bf16), wa_ref[...],
                         preferred_element_type=f32)
                 + jnp.dot(yr.astype(bf16), wb_ref[...],
                           preferred_element_type=f32))
            zbuf[pl.ds(t * m + hm * half, hm), :] = z

    @pl.when(j >= nt0 + nt1)
    def _l2():
        t = j - (nt0 + nt1)
        base = t * (m * _FANOUT)
        bias = b2_ref[...]
        for mi in range(m):
            o = base + _FANOUT * mi
            acc = (zbuf[pl.ds(idx2_ref[o], 1), :]
                   + zbuf[pl.ds(idx2_ref[o + 1], 1), :]
                   + zbuf[pl.ds(idx2_ref[o + 2], 1), :]
                   + zbuf[pl.ds(idx2_ref[o + 3], 1), :])
            o_ref[pl.ds(mi, 1), :] = acc + bias


def _gcn_call(feat, idx0, idx1, idx2, w0, b0, w1, wa, wb, b1, b2, *, m):
    n_src, fin = feat.shape
    n1 = idx0.shape[0] // _FANOUT
    n2 = idx1.shape[0] // _FANOUT
    n3 = idx2.shape[0] // _FANOUT
    fout = wa.shape[1]
    nt0, nt1, nt2 = n1 // m, n2 // m, n3 // m
    chunk = min(_CHUNK, n_src)
    kern = functools.partial(_gcn_kernel, m=m, nt0=nt0, nt1=nt1, nt2=nt2)
    s = nt0 + nt1
    return pl.pallas_call(
        kern,
        out_shape=jax.ShapeDtypeStruct((n3, fout), jnp.float32),
        grid_spec=pltpu.PrefetchScalarGridSpec(
            num_scalar_prefetch=3,
            grid=(nt0 + nt1 + nt2,),
            in_specs=[
                pl.BlockSpec(memory_space=pl.ANY),
                pl.BlockSpec(w0.shape, lambda j, i0, i1, i2: (0, 0)),
                pl.BlockSpec(b0.shape, lambda j, i0, i1, i2: (0, 0)),
                pl.BlockSpec(w1.shape, lambda j, i0, i1, i2: (0, 0)),
                pl.BlockSpec(wa.shape, lambda j, i0, i1, i2: (0, 0)),
                pl.BlockSpec(wb.shape, lambda j, i0, i1, i2: (0, 0)),
                pl.BlockSpec(b1.shape, lambda j, i0, i1, i2: (0, 0)),
                pl.BlockSpec(b2.shape, lambda j, i0, i1, i2: (0, 0)),
            ],
            out_specs=pl.BlockSpec(
                (m, fout),
                lambda j, i0, i1, i2: (jnp.maximum(j - s, 0), 0)),
            scratch_shapes=[
                pltpu.VMEM((2 * n_src, 128), jnp.float32),
                pltpu.VMEM((2 * n1, 128), jnp.float32),
                pltpu.VMEM((n2, fout), jnp.float32),
                pltpu.VMEM((chunk, fin), jnp.float32),
                pltpu.VMEM((chunk, fin), jnp.float32),
                pltpu.VMEM((chunk, fin), jnp.float32),
                pltpu.VMEM((chunk, fin), jnp.float32),
                pltpu.VMEM((m + 2, 128), jnp.float32),
                pltpu.VMEM((m + 2, 128), jnp.float32),
                pltpu.SemaphoreType.DMA((4,)),
            ],
        ),
        compiler_params=pltpu.CompilerParams(
            dimension_semantics=("arbitrary",),
            vmem_limit_bytes=58 << 20,
        ),
    )(idx0, idx1, idx2, feat, w0, b0, w1, wa, wb, b1, b2)


def kernel(features, w0, b0, w1, b1, w2, b2, nbr0, nbr1, nbr2):
    f32 = jnp.float32
    bf16 = jnp.bfloat16
    fmid = w1.shape[0]

    idx0 = (nbr0.astype(jnp.int32) * 2).reshape(-1)
    idx1 = (nbr1.astype(jnp.int32) * 2).reshape(-1)
    idx2 = nbr2.astype(jnp.int32).reshape(-1)
    w0s = (w0.astype(f32) / _FANOUT).astype(bf16)
    w1s = (w1.astype(f32) / _FANOUT).astype(bf16)
    wa = (w2[:fmid].astype(f32) / _FANOUT).astype(bf16)
    wb = (w2[fmid:].astype(f32) / _FANOUT).astype(bf16)
    out = _gcn_call(features.astype(f32), idx0, idx1, idx2,
                    w0s, b0.astype(f32).reshape(1, -1),
                    w1s, wa, wb,
                    b1.astype(f32).reshape(1, -1),
                    b2.astype(f32).reshape(1, -1), m=2048)
    return out.astype(f32)
